# Initial kernel scaffold; baseline (speedup 1.0000x reference)
import jax
import jax.numpy as jnp
from jax.experimental import pallas as pl


def kernel(node_feat, pos, vel, edge_index, params):
    raise NotImplementedError("write your pallas kernel here")


# speedup vs baseline: 2.7676x; 2.7676x over previous
import functools

import jax
import jax.numpy as jnp
import numpy as np
from jax import lax
from jax.experimental import pallas as pl
from jax.experimental.pallas import tpu as pltpu
from jax.experimental.pallas import tpu_sc as plsc

N = 10000
NP = 10240
E = 320000
DIN = 128
HID = 64
RAD = 16
SHD = 9
CUT = 5.0
PENV = 5

NC = 2
NS = 16
NW = NC * NS
CH = 80

DT = 128
DS = 128
D0 = 128
CNT = 79
DG = 24

BE = 2000
BN = 2048


def _silu(x):
    return x * jax.nn.sigmoid(x)


def _expand_deg(g):
    return jnp.concatenate(
        [g[:, 0:1],
         g[:, 1:2], g[:, 1:2], g[:, 1:2],
         g[:, 2:3], g[:, 2:3], g[:, 2:3], g[:, 2:3], g[:, 2:3]], axis=1)


def _make_gather(R, D):
    per_w = R // NW
    nch = per_w // CH
    mesh = plsc.VectorSubcoreMesh(core_axis_name="c", subcore_axis_name="s")

    @functools.partial(
        pl.kernel, mesh=mesh,
        out_type=jax.ShapeDtypeStruct((R, D), jnp.float32),
        scratch_types=[
            pltpu.VMEM((nch, CH), jnp.int32),
            pltpu.VMEM((CH, D), jnp.float32),
            pltpu.VMEM((CH, D), jnp.float32),
            pltpu.SemaphoreType.DMA,
            pltpu.SemaphoreType.DMA,
        ],
    )
    def gather_k(tab_hbm, idx_hbm, out_hbm, idx_v, buf0, buf1, sem0, sem1):
        wid = lax.axis_index("s") * NC + lax.axis_index("c")
        pltpu.sync_copy(idx_hbm.at[wid], idx_v)
        bufs = (buf0, buf1)
        sems = (sem0, sem1)
        pltpu.async_copy(tab_hbm.at[idx_v.at[0]], buf0, sem0)

        def outer(jj, carry):
            for k in range(2):
                j = jj * 2 + k
                nb = 1 - k

                @pl.when(j + 1 < nch)
                def _start():
                    pltpu.async_copy(tab_hbm.at[idx_v.at[j + 1]],
                                     bufs[nb], sems[nb])

                pltpu.make_async_copy(tab_hbm.at[pl.ds(0, CH)],
                                      bufs[k], sems[k]).wait()
                pltpu.sync_copy(bufs[k],
                                out_hbm.at[pl.ds(wid * per_w + j * CH, CH)])
            return carry

        lax.fori_loop(0, nch // 2, outer, 0)

    return gather_k


def _make_scatter(R, D):
    per_w = R // NW
    nch = per_w // CH
    rpt = NP // NS
    mesh = plsc.VectorSubcoreMesh(core_axis_name="c", subcore_axis_name="s")

    @functools.partial(
        pl.kernel, mesh=mesh,
        out_type=jax.ShapeDtypeStruct((NC, NP, D), jnp.float32),
        scratch_types=[
            pltpu.VMEM((nch, CH), jnp.int32),
            pltpu.VMEM((CH, D), jnp.float32),
            pltpu.VMEM_SHARED((NP, D), jnp.float32),
            pltpu.SemaphoreType.DMA,
        ],
    )
    def scatter_k(val_hbm, idx_hbm, zero_hbm, out_hbm, idx_v, buf, acc, sem):
        c = lax.axis_index("c")
        s = lax.axis_index("s")
        wid = s * NC + c
        pltpu.sync_copy(zero_hbm.at[pl.ds(s * rpt, rpt)],
                        acc.at[pl.ds(s * rpt, rpt)])
        pltpu.sync_copy(idx_hbm.at[wid], idx_v)
        plsc.subcore_barrier()

        def chunk(j, carry):
            pltpu.async_copy(val_hbm.at[pl.ds(wid * per_w + j * CH, CH)],
                             buf, sem).wait()
            pltpu.sync_copy(buf, acc.at[idx_v.at[j]], add=True)
            return carry

        lax.fori_loop(0, nch, chunk, 0)
        plsc.subcore_barrier()
        pltpu.sync_copy(acc.at[pl.ds(s * rpt, rpt)],
                        out_hbm.at[c, pl.ds(s * rpt, rpt)])

    return scatter_k


@functools.lru_cache(maxsize=None)
def _gather_2e_k():
    return _make_gather(2 * E, DT)


@functools.lru_cache(maxsize=None)
def _scatter_e_k():
    return _make_scatter(E, DS)


def _gather_2e(tab, idx3):
    return _gather_2e_k()(tab, idx3)


def _scatter_init(vals, idx3, zeros):
    return _scatter_e_k()(vals, idx3, zeros)


def _scatter_layer(vals, idx3, zeros):
    return _scatter_e_k()(vals, idx3, zeros)


def _full(shape):
    return pl.BlockSpec(shape, lambda i: tuple(0 for _ in shape))


def _rows(b, d):
    return pl.BlockSpec((b, d), lambda i: (i, 0))


def _tc_call(body, grid, in_specs, out_specs, out_shapes):
    return pl.pallas_call(
        body,
        grid=(grid,),
        in_specs=in_specs,
        out_specs=out_specs,
        out_shape=out_shapes,
        compiler_params=pltpu.CompilerParams(
            dimension_semantics=("arbitrary",)),
    )


def _emb_body(nf, pos, vel, W, b, out):
    h = jnp.dot(nf[...], W[...], preferred_element_type=jnp.float32) + b[...]
    z = jnp.zeros((h.shape[0], DT - HID - 6), jnp.float32)
    out[...] = jnp.concatenate([h, pos[...], vel[...], z], axis=1)


def _init_edge_body(gr, gc, W1h, W1c, W1r, b1, W2, b2, scat, geom):
    hr = gr[:, 0:HID]
    hc = gc[:, 0:HID]
    rel = gr[:, HID:HID + 3] - gc[:, HID:HID + 3]
    dv = gr[:, HID + 3:HID + 6] - gc[:, HID + 3:HID + 6]
    r2 = jnp.sum(rel * rel, axis=1, keepdims=True)
    r = jnp.sqrt(r2)
    x = r / CUT
    n = np.float32(np.pi) * (
        lax.broadcasted_iota(jnp.int32, (1, RAD), 1).astype(jnp.float32) + 1.0)
    sb = np.float32(np.sqrt(2.0 / CUT)) * jnp.sin(n * x) / (r + 1e-9)
    p = PENV
    env = (1.0 - ((p + 1) * (p + 2) / 2.0) * x ** p
           + p * (p + 2) * x ** (p + 1)
           - (p * (p + 1) / 2.0) * x ** (p + 2))
    env = jnp.where(x < 1.0, env, 0.0)
    radial = sb * env
    u = rel / (r + 1e-9)
    ux, uy, uz = u[:, 0:1], u[:, 1:2], u[:, 2:3]
    c3 = np.float32(np.sqrt(3.0))
    c15 = np.float32(np.sqrt(15.0))
    c5 = np.float32(np.sqrt(5.0))
    Y = jnp.concatenate(
        [jnp.ones_like(ux), c3 * ux, c3 * uy, c3 * uz,
         c15 * ux * uy, c15 * uy * uz, (c5 / 2.0) * (3.0 * uz * uz - 1.0),
         c15 * ux * uz, (c15 / 2.0) * (ux * ux - uy * uy)], axis=1)
    hh = _silu(jnp.dot(hr, W1h[...], preferred_element_type=jnp.float32)
               + jnp.dot(hc, W1c[...], preferred_element_type=jnp.float32)
               + jnp.dot(radial, W1r[...], preferred_element_type=jnp.float32)
               + b1[...])
    g = jnp.dot(hh, W2[...], preferred_element_type=jnp.float32) + b2[...]
    gated = Y * _expand_deg(g)
    one = jnp.ones_like(ux)
    zpad = jnp.zeros((gated.shape[0], D0 - SHD - 1), jnp.float32)
    scat[...] = jnp.concatenate([gated, one, zpad], axis=1)
    gpad = jnp.zeros((gated.shape[0], DG - 22), jnp.float32)
    geom[...] = jnp.concatenate([rel, dv, radial, gpad], axis=1)


def _init_node_body(t0, p0, p1, t1):
    ssum = p0[...] + p1[...]
    cnt = jnp.maximum(ssum[:, SHD:SHD + 1], 1.0)
    sh0 = ssum[:, 0:SHD] / cnt
    z = jnp.zeros((sh0.shape[0], DT - HID - SHD), jnp.float32)
    t1[...] = jnp.concatenate([t0[:, 0:HID], sh0, z], axis=1)


def _layer_edge_body(gr, gc, geom, W1h, W1c, W1r, W1i, b1, W2m, b2m,
                     Wp1, bp1, Wp2, bp2, Wv1, bv1, Wv2, bv2,
                     Ws1, bs1, Ws2, bs2, out):
    hr = gr[:, 0:HID]
    hc = gc[:, 0:HID]
    shr = gr[:, HID:HID + SHD]
    shc = gc[:, HID:HID + SHD]
    rel = geom[:, 0:3]
    dv = geom[:, 3:6]
    radial = geom[:, 6:6 + RAD]
    ip0 = jnp.sum(shr[:, 0:1] * shc[:, 0:1], axis=1, keepdims=True)
    ip1 = jnp.sum(shr[:, 1:4] * shc[:, 1:4], axis=1, keepdims=True)
    ip2 = jnp.sum(shr[:, 4:9] * shc[:, 4:9], axis=1, keepdims=True)
    ship = jnp.concatenate([ip0, ip1, ip2], axis=1)
    h1 = _silu(jnp.dot(hr, W1h[...], preferred_element_type=jnp.float32)
               + jnp.dot(hc, W1c[...], preferred_element_type=jnp.float32)
               + jnp.dot(radial, W1r[...], preferred_element_type=jnp.float32)
               + jnp.dot(ship, W1i[...], preferred_element_type=jnp.float32)
               + b1[...])
    msg = _silu(jnp.dot(h1, W2m[...], preferred_element_type=jnp.float32)
                + b2m[...])
    pg = jnp.dot(_silu(jnp.dot(msg, Wp1[...],
                               preferred_element_type=jnp.float32) + bp1[...]),
                 Wp2[...], preferred_element_type=jnp.float32) + bp2[...]
    vg = jnp.dot(_silu(jnp.dot(msg, Wv1[...],
                               preferred_element_type=jnp.float32) + bv1[...]),
                 Wv2[...], preferred_element_type=jnp.float32) + bv2[...]
    sg = jnp.dot(_silu(jnp.dot(msg, Ws1[...],
                               preferred_element_type=jnp.float32) + bs1[...]),
                 Ws2[...], preferred_element_type=jnp.float32) + bs2[...]
    evp = pg[:, 0:1] * rel + pg[:, 1:2] * dv
    evv = vg[:, 0:1] * dv + vg[:, 1:2] * rel
    dsh = (shr - shc) * _expand_deg(sg)
    one = jnp.ones_like(ip0)
    z = jnp.zeros((msg.shape[0], DS - CNT - 1), jnp.float32)
    out[...] = jnp.concatenate([msg, evp, evv, dsh, one, z], axis=1)


def _layer_node_body(t, p0, p1, pd, vd, Wa, Wb, b1, W2, b2,
                     tn, pdn, vdn):
    ssum = p0[...] + p1[...]
    inv = 1.0 / jnp.maximum(ssum[:, CNT:CNT + 1], 1.0)
    msg_agg = ssum[:, 0:HID] * inv
    pos_agg = ssum[:, HID:HID + 3] * inv
    vel_agg = ssum[:, HID + 3:HID + 6] * inv
    sh_agg = ssum[:, HID + 6:HID + 6 + SHD] * inv
    h = t[:, 0:HID]
    sh = t[:, HID:HID + SHD]
    hh = _silu(jnp.dot(h, Wa[...], preferred_element_type=jnp.float32)
               + jnp.dot(msg_agg, Wb[...], preferred_element_type=jnp.float32)
               + b1[...])
    hn = jnp.dot(hh, W2[...], preferred_element_type=jnp.float32) + b2[...]
    z = jnp.zeros((hn.shape[0], DT - HID - SHD), jnp.float32)
    tn[...] = jnp.concatenate([hn, sh + sh_agg, z], axis=1)
    pdn[...] = pd[...] + pos_agg
    vdn[...] = vd[...] + vel_agg


def _head_body(t, pos, vel, pd, vd, Wp1h, Wp1d, bp1, Wp2, bp2,
               Wv1h, Wv1d, Wv1v, bv1, Wv2, bv2, out):
    h = t[:, 0:HID]
    ph = _silu(jnp.dot(h, Wp1h[...], preferred_element_type=jnp.float32)
               + jnp.dot(pd[...], Wp1d[...],
                         preferred_element_type=jnp.float32) + bp1[...])
    pos_out = pos[...] + (jnp.dot(ph, Wp2[...],
                                  preferred_element_type=jnp.float32)
                          + bp2[...])
    vh = _silu(jnp.dot(h, Wv1h[...], preferred_element_type=jnp.float32)
               + jnp.dot(vd[...], Wv1d[...],
                         preferred_element_type=jnp.float32)
               + jnp.dot(vel[...], Wv1v[...],
                         preferred_element_type=jnp.float32) + bv1[...])
    vel_out = (jnp.dot(vh, Wv2[...], preferred_element_type=jnp.float32)
               + bv2[...])
    out[...] = jnp.concatenate([pos_out, vel_out], axis=1)


def _b2(b):
    return b.reshape(1, -1)


def kernel(node_feat, pos, vel, edge_index, params):
    f32 = jnp.float32
    npad = NP - N
    nf = jnp.pad(node_feat.astype(f32), ((0, npad), (0, 0)))
    posp = jnp.pad(pos.astype(f32), ((0, npad), (0, 0)))
    velp = jnp.pad(vel.astype(f32), ((0, npad), (0, 0)))

    row = edge_index[0].astype(jnp.int32)
    idx_all = edge_index.astype(jnp.int32).reshape(NW, (2 * E) // NW // CH, CH)
    idx_row = row.reshape(NW, E // NW // CH, CH)

    zero_n0 = jnp.zeros((NP, D0), f32)
    zero_ns = jnp.zeros((NP, DS), f32)

    egrid = E // BE
    ngrid = NP // BN

    t0 = _tc_call(
        _emb_body, ngrid,
        [_rows(BN, DIN), _rows(BN, 3), _rows(BN, 3),
         _full((DIN, HID)), _full((1, HID))],
        _rows(BN, DT),
        jax.ShapeDtypeStruct((NP, DT), f32),
    )(nf, posp, velp, params['emb_W'], _b2(params['emb_b']))

    g = _gather_2e(t0, idx_all)
    gr0, gc0 = g[:E], g[E:]
    w = params['sh_init']
    scat0, geom = _tc_call(
        _init_edge_body, egrid,
        [_rows(BE, DT), _rows(BE, DT),
         _full((HID, HID)), _full((HID, HID)), _full((RAD, HID)),
         _full((1, HID)), _full((HID, 3)), _full((1, 3))],
        [_rows(BE, D0), _rows(BE, DG)],
        [jax.ShapeDtypeStruct((E, D0), f32),
         jax.ShapeDtypeStruct((E, DG), f32)],
    )(gr0, gc0, w['W1'][0:HID], w['W1'][HID:2 * HID], w['W1'][2 * HID:],
      _b2(w['b1']), w['W2'], _b2(w['b2']))

    p = _scatter_init(scat0, idx_row, zero_n0)
    t = _tc_call(
        _init_node_body, ngrid,
        [_rows(BN, DT), _rows(BN, D0), _rows(BN, D0)],
        _rows(BN, DT),
        jax.ShapeDtypeStruct((NP, DT), f32),
    )(t0, p[0], p[1])

    pd = jnp.zeros((NP, 3), f32)
    vd = jnp.zeros((NP, 3), f32)

    for lp in params['layers']:
        g = _gather_2e(t, idx_all)
        gr, gc = g[:E], g[E:]
        m = lp['msg']
        s = _tc_call(
            _layer_edge_body, egrid,
            [_rows(BE, DT), _rows(BE, DT), _rows(BE, DG),
             _full((HID, HID)), _full((HID, HID)), _full((RAD, HID)),
             _full((3, HID)), _full((1, HID)),
             _full((HID, HID)), _full((1, HID)),
             _full((HID, HID)), _full((1, HID)), _full((HID, 2)),
             _full((1, 2)),
             _full((HID, HID)), _full((1, HID)), _full((HID, 2)),
             _full((1, 2)),
             _full((HID, HID)), _full((1, HID)), _full((HID, 3)),
             _full((1, 3))],
            _rows(BE, DS),
            jax.ShapeDtypeStruct((E, DS), f32),
        )(gr, gc, geom,
          m['W1'][0:HID], m['W1'][HID:2 * HID],
          m['W1'][2 * HID:2 * HID + RAD], m['W1'][2 * HID + RAD:],
          _b2(m['b1']), m['W2'], _b2(m['b2']),
          lp['pos_basis']['W1'], _b2(lp['pos_basis']['b1']),
          lp['pos_basis']['W2'], _b2(lp['pos_basis']['b2']),
          lp['vel_basis']['W1'], _b2(lp['vel_basis']['b1']),
          lp['vel_basis']['W2'], _b2(lp['vel_basis']['b2']),
          lp['mlp_sh']['W1'], _b2(lp['mlp_sh']['b1']),
          lp['mlp_sh']['W2'], _b2(lp['mlp_sh']['b2']))

        p = _scatter_layer(s, idx_row, zero_ns)
        nw = lp['node_feat']
        t, pd, vd = _tc_call(
            _layer_node_body, ngrid,
            [_rows(BN, DT), _rows(BN, DS), _rows(BN, DS),
             _rows(BN, 3), _rows(BN, 3),
             _full((HID, HID)), _full((HID, HID)), _full((1, HID)),
             _full((HID, HID)), _full((1, HID))],
            [_rows(BN, DT), _rows(BN, 3), _rows(BN, 3)],
            [jax.ShapeDtypeStruct((NP, DT), f32),
             jax.ShapeDtypeStruct((NP, 3), f32),
             jax.ShapeDtypeStruct((NP, 3), f32)],
        )(t, p[0], p[1], pd, vd,
          nw['W1'][0:HID], nw['W1'][HID:], _b2(nw['b1']),
          nw['W2'], _b2(nw['b2']))

    ph = params['pos_head']
    vh = params['vel_head']
    out = _tc_call(
        _head_body, ngrid,
        [_rows(BN, DT), _rows(BN, 3), _rows(BN, 3),
         _rows(BN, 3), _rows(BN, 3),
         _full((HID, HID)), _full((3, HID)), _full((1, HID)),
         _full((HID, 3)), _full((1, 3)),
         _full((HID, HID)), _full((3, HID)), _full((3, HID)),
         _full((1, HID)), _full((HID, 3)), _full((1, 3))],
        _rows(BN, 6),
        jax.ShapeDtypeStruct((NP, 6), f32),
    )(t, posp, velp, pd, vd,
      ph['W1'][0:HID], ph['W1'][HID:], _b2(ph['b1']), ph['W2'], _b2(ph['b2']),
      vh['W1'][0:HID], vh['W1'][HID:HID + 3], vh['W1'][HID + 3:],
      _b2(vh['b1']), vh['W2'], _b2(vh['b2']))

    return out[:N]


# speedup vs baseline: 5.1275x; 1.8527x over previous
import functools

import jax
import jax.numpy as jnp
import numpy as np
from jax import lax
from jax.experimental import pallas as pl
from jax.experimental.pallas import tpu as pltpu
from jax.experimental.pallas import tpu_sc as plsc

N = 10000
NP = 10240
E = 320000
DIN = 128
HID = 64
RAD = 16
SHD = 9
CUT = 5.0
PENV = 5

NC = 2
NS = 16
NW = NC * NS
CH = 80

DT = 128
DS = 128
D0 = 128
CNT = 79
DG = 24

BE = 6400
BN = 2048

E2 = E // 2
CHS = 40


def _silu(x):
    return x * jax.nn.sigmoid(x)


def _expand_mat():
    ii = lax.broadcasted_iota(jnp.int32, (3, SHD), 0)
    jj = lax.broadcasted_iota(jnp.int32, (3, SHD), 1)
    return ((jj >= ii * ii) & (jj < (ii + 1) * (ii + 1))).astype(jnp.float32)


def _expand_deg(g):
    return jnp.dot(g, _expand_mat(), preferred_element_type=jnp.float32)


def _dotT(at, w):
    return lax.dot_general(at, w, (((0,), (0,)), ((), ())),
                           preferred_element_type=jnp.float32)


def _eye(k):
    return (lax.broadcasted_iota(jnp.int32, (k, k), 0) ==
            lax.broadcasted_iota(jnp.int32, (k, k), 1)).astype(jnp.float32)


def _t_rows(a, k):
    return lax.dot_general(_eye(k), a, (((1,), (1,)), ((), ())),
                           preferred_element_type=jnp.float32)


def _t_cols(at, k):
    return lax.dot_general(at, _eye(k), (((0,), (0,)), ((), ())),
                           preferred_element_type=jnp.float32)


def _make_gather(R, D, ch=CH):
    per_w = R // NW
    nch = per_w // ch
    nsup = nch // 2
    SUP = 2 * ch
    mesh = plsc.VectorSubcoreMesh(core_axis_name="c", subcore_axis_name="s")

    @functools.partial(
        pl.kernel, mesh=mesh,
        out_type=jax.ShapeDtypeStruct((R, D), jnp.float32),
        scratch_types=[
            pltpu.VMEM((nch, ch), jnp.int32),
            pltpu.VMEM((SUP, D), jnp.float32),
            pltpu.VMEM((SUP, D), jnp.float32),
            pltpu.SemaphoreType.DMA,
            pltpu.SemaphoreType.DMA,
            pltpu.SemaphoreType.DMA,
            pltpu.SemaphoreType.DMA,
        ],
    )
    def gather_k(tab_hbm, idx_hbm, out_hbm, idx_v,
                 buf0, buf1, in0, in1, ou0, ou1):
        wid = lax.axis_index("s") * NC + lax.axis_index("c")
        pltpu.sync_copy(idx_hbm.at[wid], idx_v)
        bufs = (buf0, buf1)
        ins = (in0, in1)
        ous = (ou0, ou1)

        def start_super(s, b):
            for h in range(2):
                pltpu.async_copy(tab_hbm.at[idx_v.at[2 * s + h]],
                                 bufs[b].at[pl.ds(h * ch, ch)], ins[b])

        def wait_in(b):
            for h in range(2):
                pltpu.make_async_copy(tab_hbm.at[pl.ds(0, ch)],
                                      bufs[b].at[pl.ds(0, ch)],
                                      ins[b]).wait()

        def wait_out(b):
            pltpu.make_async_copy(bufs[b], out_hbm.at[pl.ds(0, SUP)],
                                  ous[b]).wait()

        start_super(0, 0)

        def outer(jj, carry):
            for k in range(2):
                s = jj * 2 + k
                nb = 1 - k

                @pl.when(s + 1 < nsup)
                def _ahead():
                    @pl.when(s >= 1)
                    def _drain():
                        wait_out(nb)

                    start_super(s + 1, nb)

                wait_in(k)
                pltpu.async_copy(
                    bufs[k], out_hbm.at[pl.ds(wid * per_w + s * SUP, SUP)],
                    ous[k])
            return carry

        lax.fori_loop(0, nsup // 2, outer, 0)
        if nsup % 2 == 1:
            wait_in(0)
            pltpu.async_copy(
                bufs[0],
                out_hbm.at[pl.ds(wid * per_w + (nsup - 1) * SUP, SUP)],
                ous[0])
        wait_out(1 - (nsup % 2))
        wait_out(nsup % 2)

    return gather_k


def _make_scatter(R, D, ch=CH):
    per_w = R // NW
    nch = per_w // ch
    rpt = NP // NS
    mesh = plsc.VectorSubcoreMesh(core_axis_name="c", subcore_axis_name="s")

    @functools.partial(
        pl.kernel, mesh=mesh,
        out_type=jax.ShapeDtypeStruct((NC, NP, D), jnp.float32),
        scratch_types=[
            pltpu.VMEM((nch, ch), jnp.int32),
            pltpu.VMEM((ch, D), jnp.float32),
            pltpu.VMEM((ch, D), jnp.float32),
            pltpu.VMEM_SHARED((NP, D), jnp.float32),
            pltpu.SemaphoreType.DMA,
            pltpu.SemaphoreType.DMA,
            pltpu.SemaphoreType.DMA,
            pltpu.SemaphoreType.DMA,
        ],
    )
    def scatter_k(val_hbm, idx_hbm, zero_hbm, out_hbm, idx_v,
                  buf0, buf1, acc, ld0, ld1, ad0, ad1):
        c = lax.axis_index("c")
        s = lax.axis_index("s")
        wid = s * NC + c
        pltpu.sync_copy(zero_hbm.at[pl.ds(s * rpt, rpt)],
                        acc.at[pl.ds(s * rpt, rpt)])
        pltpu.sync_copy(idx_hbm.at[wid], idx_v)
        plsc.subcore_barrier()
        bufs = (buf0, buf1)
        lds = (ld0, ld1)
        ads = (ad0, ad1)

        def load(j, b):
            pltpu.async_copy(val_hbm.at[pl.ds(wid * per_w + j * ch, ch)],
                             bufs[b], lds[b])

        def wait_ld(b):
            pltpu.make_async_copy(val_hbm.at[pl.ds(0, ch)], bufs[b],
                                  lds[b]).wait()

        def wait_add(b):
            pltpu.make_async_copy(val_hbm.at[pl.ds(0, ch)], bufs[b],
                                  ads[b]).wait()

        load(0, 0)

        def outer(jj, carry):
            for k in range(2):
                j = jj * 2 + k
                nb = 1 - k

                @pl.when(j + 1 < nch)
                def _ahead():
                    @pl.when(j >= 1)
                    def _drain():
                        wait_add(nb)

                    load(j + 1, nb)

                wait_ld(k)
                pltpu.async_copy(bufs[k], acc.at[idx_v.at[j]], ads[k],
                                 add=True)
            return carry

        lax.fori_loop(0, nch // 2, outer, 0)
        if nch % 2 == 1:
            wait_ld(0)
            pltpu.async_copy(bufs[0], acc.at[idx_v.at[nch - 1]], ads[0],
                             add=True)
        wait_add(1 - (nch % 2))
        wait_add(nch % 2)
        plsc.subcore_barrier()
        pltpu.sync_copy(acc.at[pl.ds(s * rpt, rpt)],
                        out_hbm.at[c, pl.ds(s * rpt, rpt)])

    return scatter_k


@functools.lru_cache(maxsize=None)
def _gather_half_k():
    return _make_gather(E, DT, CHS)


@functools.lru_cache(maxsize=None)
def _scatter_half_k():
    return _make_scatter(E2, DS, CHS)


def _gather_half(tab, idx3):
    return _gather_half_k()(tab, idx3)


def _scatter_half(vals, idx3, zeros):
    return _scatter_half_k()(vals, idx3, zeros)


def _full(shape):
    return pl.BlockSpec(shape, lambda i: tuple(0 for _ in shape))


def _rows(b, d):
    return pl.BlockSpec((b, d), lambda i: (i, 0))


def _tc_call(body, grid, in_specs, out_specs, out_shapes):
    return pl.pallas_call(
        body,
        grid=(grid,),
        in_specs=in_specs,
        out_specs=out_specs,
        out_shape=out_shapes,
        compiler_params=pltpu.CompilerParams(
            dimension_semantics=("arbitrary",)),
    )


def _emb_body(nf, pos, vel, W, b, out):
    h = jnp.dot(nf[...], W[...], preferred_element_type=jnp.float32) + b[...]
    z = jnp.zeros((h.shape[0], DT - HID - 6), jnp.float32)
    out[...] = jnp.concatenate([h, pos[...], vel[...], z], axis=1)


def _init_edge_body(gr, gc, W1h, W1c, W1r, b1, W2, b2, scat, geom):
    be = gr.shape[0]
    hr = gr[:, 0:HID]
    hc = gc[:, 0:HID]
    mT = (gr[:, HID:HID + 8] - gc[:, HID:HID + 8]).T
    relT = mT[0:3]
    dvT = mT[3:6]
    r2T = relT[0:1] * relT[0:1] + relT[1:2] * relT[1:2] + relT[2:3] * relT[2:3]
    rT = jnp.sqrt(r2T)
    xT = rT * np.float32(1.0 / CUT)
    nT = np.float32(np.pi) * (
        lax.broadcasted_iota(jnp.int32, (RAD, 1), 0).astype(jnp.float32)
        + 1.0)
    sbT = (np.float32(np.sqrt(2.0 / CUT)) * jnp.sin(nT * xT)
           / (rT + 1e-9))
    p = PENV
    envT = (1.0 - ((p + 1) * (p + 2) / 2.0) * xT ** p
            + p * (p + 2) * xT ** (p + 1)
            - (p * (p + 1) / 2.0) * xT ** (p + 2))
    envT = jnp.where(xT < 1.0, envT, 0.0)
    radialT = sbT * envT
    uT = relT / (rT + 1e-9)
    ux, uy, uz = uT[0:1], uT[1:2], uT[2:3]
    c3 = np.float32(np.sqrt(3.0))
    c15 = np.float32(np.sqrt(15.0))
    c5 = np.float32(np.sqrt(5.0))
    YT = jnp.concatenate(
        [jnp.ones_like(ux), c3 * ux, c3 * uy, c3 * uz,
         c15 * ux * uy, c15 * uy * uz, (c5 / 2.0) * (3.0 * uz * uz - 1.0),
         c15 * ux * uz, (c15 / 2.0) * (ux * ux - uy * uy)], axis=0)
    hh = _silu(jnp.dot(hr, W1h[...], preferred_element_type=jnp.float32)
               + jnp.dot(hc, W1c[...], preferred_element_type=jnp.float32)
               + _dotT(radialT, W1r[...])
               + b1[...])
    g = jnp.dot(hh, W2[...], preferred_element_type=jnp.float32) + b2[...]
    egT = lax.dot_general(_expand_mat(), g, (((0,), (1,)), ((), ())),
                          preferred_element_type=jnp.float32)
    sT = jnp.concatenate([YT * egT, jnp.ones((1, be), jnp.float32)], axis=0)
    scat[:, 0:SHD + 1] = sT.T
    scat[:, SHD + 1:] = jnp.zeros((be, D0 - SHD - 1), jnp.float32)
    geom[...] = jnp.concatenate(
        [mT[0:6], radialT, jnp.zeros((DG - 22, be), jnp.float32)], axis=0)


def _init_node_body(t0, p0, p1, p2, p3, t1):
    ssum = p0[...] + p1[...] + p2[...] + p3[...]
    cnt = jnp.maximum(ssum[:, SHD:SHD + 1], 1.0)
    sh0 = ssum[:, 0:SHD] / cnt
    z = jnp.zeros((sh0.shape[0], DT - HID - SHD), jnp.float32)
    t1[...] = jnp.concatenate([t0[:, 0:HID], sh0, z], axis=1)


def _layer_edge_body(gr, gc, geom, W1h, W1c, W1r, W1i, b1, W2m, b2m,
                     Wg1, bg1, Wg2, bg2, out):
    be = gr.shape[0]
    hr = gr[:, 0:HID]
    hc = gc[:, 0:HID]
    shrT = _t_rows(gr[:, HID:HID + SHD], SHD)
    shcT = _t_rows(gc[:, HID:HID + SHD], SHD)
    relT = geom[0:3]
    dvT = geom[3:6]
    radialT = geom[6:6 + RAD]
    prodT = shrT * shcT
    ip0 = prodT[0:1]
    ip1 = prodT[1:2] + prodT[2:3] + prodT[3:4]
    ip2 = (prodT[4:5] + prodT[5:6] + prodT[6:7] + prodT[7:8] + prodT[8:9])
    shipT = jnp.concatenate([ip0, ip1, ip2], axis=0)
    h1 = _silu(jnp.dot(hr, W1h[...], preferred_element_type=jnp.float32)
               + jnp.dot(hc, W1c[...], preferred_element_type=jnp.float32)
               + _dotT(radialT, W1r[...])
               + _dotT(shipT, W1i[...])
               + b1[...])
    msg = _silu(jnp.dot(h1, W2m[...], preferred_element_type=jnp.float32)
                + b2m[...])
    hidg = _silu(jnp.dot(msg, Wg1[...], preferred_element_type=jnp.float32)
                 + bg1[...])
    g_all = jnp.dot(hidg, Wg2[...], preferred_element_type=jnp.float32) \
        + bg2[...]
    gT = _t_rows(g_all, 7)
    evpT = gT[0:1] * relT + gT[1:2] * dvT
    evvT = gT[2:3] * dvT + gT[3:4] * relT
    egT = lax.dot_general(_expand_mat(), gT[4:7], (((0,), (0,)), ((), ())),
                          preferred_element_type=jnp.float32)
    dshT = (shrT - shcT) * egT
    tailT = jnp.concatenate(
        [evpT, evvT, dshT, jnp.ones((1, be), jnp.float32)], axis=0)
    out[:, 0:HID] = msg
    out[:, HID:CNT + 1] = _t_cols(tailT, CNT + 1 - HID)
    out[:, CNT + 1:] = jnp.zeros((be, DS - CNT - 1), jnp.float32)


def _layer_node_body(t, p0, p1, p2, p3, pd, vd, Wa, Wb, b1, W2, b2,
                     tn, pdn, vdn):
    ssum = p0[...] + p1[...] + p2[...] + p3[...]
    inv = 1.0 / jnp.maximum(ssum[:, CNT:CNT + 1], 1.0)
    msg_agg = ssum[:, 0:HID] * inv
    pos_agg = ssum[:, HID:HID + 3] * inv
    vel_agg = ssum[:, HID + 3:HID + 6] * inv
    sh_agg = ssum[:, HID + 6:HID + 6 + SHD] * inv
    h = t[:, 0:HID]
    sh = t[:, HID:HID + SHD]
    hh = _silu(jnp.dot(h, Wa[...], preferred_element_type=jnp.float32)
               + jnp.dot(msg_agg, Wb[...], preferred_element_type=jnp.float32)
               + b1[...])
    hn = jnp.dot(hh, W2[...], preferred_element_type=jnp.float32) + b2[...]
    z = jnp.zeros((hn.shape[0], DT - HID - SHD), jnp.float32)
    tn[...] = jnp.concatenate([hn, sh + sh_agg, z], axis=1)
    pdn[...] = pd[...] + pos_agg
    vdn[...] = vd[...] + vel_agg


def _head_body(t, pos, vel, pd, vd, Wp1h, Wp1d, bp1, Wp2, bp2,
               Wv1h, Wv1d, Wv1v, bv1, Wv2, bv2, out):
    h = t[:, 0:HID]
    ph = _silu(jnp.dot(h, Wp1h[...], preferred_element_type=jnp.float32)
               + jnp.dot(pd[...], Wp1d[...],
                         preferred_element_type=jnp.float32) + bp1[...])
    pos_out = pos[...] + (jnp.dot(ph, Wp2[...],
                                  preferred_element_type=jnp.float32)
                          + bp2[...])
    vh = _silu(jnp.dot(h, Wv1h[...], preferred_element_type=jnp.float32)
               + jnp.dot(vd[...], Wv1d[...],
                         preferred_element_type=jnp.float32)
               + jnp.dot(vel[...], Wv1v[...],
                         preferred_element_type=jnp.float32) + bv1[...])
    vel_out = (jnp.dot(vh, Wv2[...], preferred_element_type=jnp.float32)
               + bv2[...])
    out[...] = jnp.concatenate([pos_out, vel_out], axis=1)


def _b2(b):
    return b.reshape(1, -1)


def kernel(node_feat, pos, vel, edge_index, params):
    f32 = jnp.float32
    npad = NP - N
    nf = jnp.pad(node_feat.astype(f32), ((0, npad), (0, 0)))
    posp = jnp.pad(pos.astype(f32), ((0, npad), (0, 0)))
    velp = jnp.pad(vel.astype(f32), ((0, npad), (0, 0)))

    ei = edge_index.astype(jnp.int32)
    row = ei[0]
    idx_g = [ei[:, h * E2:(h + 1) * E2].reshape(NW, E // NW // CHS, CHS)
             for h in (0, 1)]
    idx_s = [row[h * E2:(h + 1) * E2].reshape(NW, E2 // NW // CHS, CHS)
             for h in (0, 1)]

    zero_n0 = jnp.zeros((NP, D0), f32)
    zero_ns = jnp.zeros((NP, DS), f32)

    egrid = E2 // BE
    ngrid = NP // BN

    t0 = _tc_call(
        _emb_body, ngrid,
        [_rows(BN, DIN), _rows(BN, 3), _rows(BN, 3),
         _full((DIN, HID)), _full((1, HID))],
        _rows(BN, DT),
        jax.ShapeDtypeStruct((NP, DT), f32),
    )(nf, posp, velp, params['emb_W'], _b2(params['emb_b']))

    w = params['sh_init']

    def init_edge(gh):
        return _tc_call(
            _init_edge_body, egrid,
            [_rows(BE, DT), _rows(BE, DT),
             _full((HID, HID)), _full((HID, HID)), _full((RAD, HID)),
             _full((1, HID)), _full((HID, 3)), _full((1, 3))],
            [_rows(BE, D0), pl.BlockSpec((DG, BE), lambda i: (0, i))],
            [jax.ShapeDtypeStruct((E2, D0), f32),
             jax.ShapeDtypeStruct((DG, E2), f32)],
        )(gh[:E2], gh[E2:], w['W1'][0:HID], w['W1'][HID:2 * HID],
          w['W1'][2 * HID:], _b2(w['b1']), w['W2'], _b2(w['b2']))

    g0 = _gather_half(t0, idx_g[0])
    scat0a, geom0 = init_edge(g0)
    g1 = _gather_half(t0, idx_g[1])
    scat0b, geom1 = init_edge(g1)
    pa = _scatter_half(scat0a, idx_s[0], zero_n0)
    pb = _scatter_half(scat0b, idx_s[1], zero_n0)
    t = _tc_call(
        _init_node_body, ngrid,
        [_rows(BN, DT), _rows(BN, D0), _rows(BN, D0),
         _rows(BN, D0), _rows(BN, D0)],
        _rows(BN, DT),
        jax.ShapeDtypeStruct((NP, DT), f32),
    )(t0, pa[0], pa[1], pb[0], pb[1])

    pd = jnp.zeros((NP, 3), f32)
    vd = jnp.zeros((NP, 3), f32)

    for lp in params['layers']:
        m = lp['msg']
        Wg1 = jnp.concatenate([lp['pos_basis']['W1'], lp['vel_basis']['W1'],
                               lp['mlp_sh']['W1']], axis=1)
        bg1 = jnp.concatenate([lp['pos_basis']['b1'], lp['vel_basis']['b1'],
                               lp['mlp_sh']['b1']])
        z64 = jnp.zeros((HID, 2), jnp.float32)
        z64b = jnp.zeros((HID, 3), jnp.float32)
        Wg2 = jnp.concatenate([
            jnp.concatenate([lp['pos_basis']['W2'], z64, z64b], axis=1),
            jnp.concatenate([z64, lp['vel_basis']['W2'], z64b], axis=1),
            jnp.concatenate([z64, z64, lp['mlp_sh']['W2']], axis=1),
        ], axis=0)
        bg2 = jnp.concatenate([lp['pos_basis']['b2'], lp['vel_basis']['b2'],
                               lp['mlp_sh']['b2']])
        def layer_edge(gh, geomh):
            return _tc_call(
                _layer_edge_body, egrid,
                [_rows(BE, DT), _rows(BE, DT),
                 pl.BlockSpec((DG, BE), lambda i: (0, i)),
                 _full((HID, HID)), _full((HID, HID)), _full((RAD, HID)),
                 _full((3, HID)), _full((1, HID)),
                 _full((HID, HID)), _full((1, HID)),
                 _full((HID, 3 * HID)), _full((1, 3 * HID)),
                 _full((3 * HID, 7)), _full((1, 7))],
                _rows(BE, DS),
                jax.ShapeDtypeStruct((E2, DS), f32),
            )(gh[:E2], gh[E2:], geomh,
              m['W1'][0:HID], m['W1'][HID:2 * HID],
              m['W1'][2 * HID:2 * HID + RAD], m['W1'][2 * HID + RAD:],
              _b2(m['b1']), m['W2'], _b2(m['b2']),
              Wg1, _b2(bg1), Wg2, _b2(bg2))

        g0 = _gather_half(t, idx_g[0])
        s0 = layer_edge(g0, geom0)
        g1 = _gather_half(t, idx_g[1])
        s1 = layer_edge(g1, geom1)
        pa = _scatter_half(s0, idx_s[0], zero_ns)
        pb = _scatter_half(s1, idx_s[1], zero_ns)
        nw = lp['node_feat']
        t, pd, vd = _tc_call(
            _layer_node_body, ngrid,
            [_rows(BN, DT), _rows(BN, DS), _rows(BN, DS),
             _rows(BN, DS), _rows(BN, DS),
             _rows(BN, 3), _rows(BN, 3),
             _full((HID, HID)), _full((HID, HID)), _full((1, HID)),
             _full((HID, HID)), _full((1, HID))],
            [_rows(BN, DT), _rows(BN, 3), _rows(BN, 3)],
            [jax.ShapeDtypeStruct((NP, DT), f32),
             jax.ShapeDtypeStruct((NP, 3), f32),
             jax.ShapeDtypeStruct((NP, 3), f32)],
        )(t, pa[0], pa[1], pb[0], pb[1], pd, vd,
          nw['W1'][0:HID], nw['W1'][HID:], _b2(nw['b1']),
          nw['W2'], _b2(nw['b2']))

    ph = params['pos_head']
    vh = params['vel_head']
    out = _tc_call(
        _head_body, ngrid,
        [_rows(BN, DT), _rows(BN, 3), _rows(BN, 3),
         _rows(BN, 3), _rows(BN, 3),
         _full((HID, HID)), _full((3, HID)), _full((1, HID)),
         _full((HID, 3)), _full((1, 3)),
         _full((HID, HID)), _full((3, HID)), _full((3, HID)),
         _full((1, HID)), _full((HID, 3)), _full((1, 3))],
        _rows(BN, 6),
        jax.ShapeDtypeStruct((NP, 6), f32),
    )(t, posp, velp, pd, vd,
      ph['W1'][0:HID], ph['W1'][HID:], _b2(ph['b1']), ph['W2'], _b2(ph['b2']),
      vh['W1'][0:HID], vh['W1'][HID:HID + 3], vh['W1'][HID + 3:],
      _b2(vh['b1']), vh['W2'], _b2(vh['b2']))

    return out[:N]


# speedup vs baseline: 5.3089x; 1.0354x over previous
import functools

import jax
import jax.numpy as jnp
import numpy as np
from jax import lax
from jax.experimental import pallas as pl
from jax.experimental.pallas import tpu as pltpu
from jax.experimental.pallas import tpu_sc as plsc

N = 10000
NP = 10240
E = 320000
DIN = 128
HID = 64
RAD = 16
SHD = 9
CUT = 5.0
PENV = 5

NC = 2
NS = 16
NW = NC * NS
CH = 80

DT = 128
DS = 128
D0 = 128
CNT = 79
DG = 24

BE = 6400
BN = 2048

SPLITS = (128000, 192000)
OFFS = (0, 128000)


def _silu(x):
    return x * jax.nn.sigmoid(x)


def _expand_mat():
    ii = lax.broadcasted_iota(jnp.int32, (3, SHD), 0)
    jj = lax.broadcasted_iota(jnp.int32, (3, SHD), 1)
    return ((jj >= ii * ii) & (jj < (ii + 1) * (ii + 1))).astype(jnp.float32)


def _expand_deg(g):
    return jnp.dot(g, _expand_mat(), preferred_element_type=jnp.float32)


def _dotT(at, w):
    return lax.dot_general(at, w, (((0,), (0,)), ((), ())),
                           preferred_element_type=jnp.float32)


def _eye(k):
    return (lax.broadcasted_iota(jnp.int32, (k, k), 0) ==
            lax.broadcasted_iota(jnp.int32, (k, k), 1)).astype(jnp.float32)


def _t_rows(a, k):
    return lax.dot_general(_eye(k), a, (((1,), (1,)), ((), ())),
                           preferred_element_type=jnp.float32)


def _t_cols(at, k):
    return lax.dot_general(at, _eye(k), (((0,), (0,)), ((), ())),
                           preferred_element_type=jnp.float32)


def _make_gather(R, D, ch=CH):
    per_w = R // NW
    nch = per_w // ch
    nsup = nch // 2
    SUP = 2 * ch
    mesh = plsc.VectorSubcoreMesh(core_axis_name="c", subcore_axis_name="s")

    @functools.partial(
        pl.kernel, mesh=mesh,
        out_type=jax.ShapeDtypeStruct((R, D), jnp.float32),
        scratch_types=[
            pltpu.VMEM((nch, ch), jnp.int32),
            pltpu.VMEM((SUP, D), jnp.float32),
            pltpu.VMEM((SUP, D), jnp.float32),
            pltpu.SemaphoreType.DMA,
            pltpu.SemaphoreType.DMA,
            pltpu.SemaphoreType.DMA,
            pltpu.SemaphoreType.DMA,
        ],
    )
    def gather_k(tab_hbm, idx_hbm, out_hbm, idx_v,
                 buf0, buf1, in0, in1, ou0, ou1):
        wid = lax.axis_index("s") * NC + lax.axis_index("c")
        pltpu.sync_copy(idx_hbm.at[wid], idx_v)
        bufs = (buf0, buf1)
        ins = (in0, in1)
        ous = (ou0, ou1)

        def start_super(s, b):
            for h in range(2):
                pltpu.async_copy(tab_hbm.at[idx_v.at[2 * s + h]],
                                 bufs[b].at[pl.ds(h * ch, ch)], ins[b])

        def wait_in(b):
            for h in range(2):
                pltpu.make_async_copy(tab_hbm.at[pl.ds(0, ch)],
                                      bufs[b].at[pl.ds(0, ch)],
                                      ins[b]).wait()

        def wait_out(b):
            pltpu.make_async_copy(bufs[b], out_hbm.at[pl.ds(0, SUP)],
                                  ous[b]).wait()

        start_super(0, 0)

        def outer(jj, carry):
            for k in range(2):
                s = jj * 2 + k
                nb = 1 - k

                @pl.when(s + 1 < nsup)
                def _ahead():
                    @pl.when(s >= 1)
                    def _drain():
                        wait_out(nb)

                    start_super(s + 1, nb)

                wait_in(k)
                pltpu.async_copy(
                    bufs[k], out_hbm.at[pl.ds(wid * per_w + s * SUP, SUP)],
                    ous[k])
            return carry

        lax.fori_loop(0, nsup // 2, outer, 0)
        if nsup % 2 == 1:
            wait_in(0)
            pltpu.async_copy(
                bufs[0],
                out_hbm.at[pl.ds(wid * per_w + (nsup - 1) * SUP, SUP)],
                ous[0])
        wait_out(1 - (nsup % 2))
        wait_out(nsup % 2)

    return gather_k


def _make_scatter(R, D, ch=CH):
    per_w = R // NW
    nch = per_w // ch
    rpt = NP // NS
    mesh = plsc.VectorSubcoreMesh(core_axis_name="c", subcore_axis_name="s")

    @functools.partial(
        pl.kernel, mesh=mesh,
        out_type=jax.ShapeDtypeStruct((NC, NP, D), jnp.float32),
        scratch_types=[
            pltpu.VMEM((nch, ch), jnp.int32),
            pltpu.VMEM((ch, D), jnp.float32),
            pltpu.VMEM((ch, D), jnp.float32),
            pltpu.VMEM_SHARED((NP, D), jnp.float32),
            pltpu.SemaphoreType.DMA,
            pltpu.SemaphoreType.DMA,
            pltpu.SemaphoreType.DMA,
            pltpu.SemaphoreType.DMA,
        ],
    )
    def scatter_k(val_hbm, idx_hbm, zero_hbm, out_hbm, idx_v,
                  buf0, buf1, acc, ld0, ld1, ad0, ad1):
        c = lax.axis_index("c")
        s = lax.axis_index("s")
        wid = s * NC + c
        pltpu.sync_copy(zero_hbm.at[pl.ds(s * rpt, rpt)],
                        acc.at[pl.ds(s * rpt, rpt)])
        pltpu.sync_copy(idx_hbm.at[wid], idx_v)
        plsc.subcore_barrier()
        bufs = (buf0, buf1)
        lds = (ld0, ld1)
        ads = (ad0, ad1)

        def load(j, b):
            pltpu.async_copy(val_hbm.at[pl.ds(wid * per_w + j * ch, ch)],
                             bufs[b], lds[b])

        def wait_ld(b):
            pltpu.make_async_copy(val_hbm.at[pl.ds(0, ch)], bufs[b],
                                  lds[b]).wait()

        def wait_add(b):
            pltpu.make_async_copy(val_hbm.at[pl.ds(0, ch)], bufs[b],
                                  ads[b]).wait()

        load(0, 0)

        def outer(jj, carry):
            for k in range(2):
                j = jj * 2 + k
                nb = 1 - k

                @pl.when(j + 1 < nch)
                def _ahead():
                    @pl.when(j >= 1)
                    def _drain():
                        wait_add(nb)

                    load(j + 1, nb)

                wait_ld(k)
                pltpu.async_copy(bufs[k], acc.at[idx_v.at[j]], ads[k],
                                 add=True)
            return carry

        lax.fori_loop(0, nch // 2, outer, 0)
        if nch % 2 == 1:
            wait_ld(0)
            pltpu.async_copy(bufs[0], acc.at[idx_v.at[nch - 1]], ads[0],
                             add=True)
        wait_add(1 - (nch % 2))
        wait_add(nch % 2)
        plsc.subcore_barrier()
        pltpu.sync_copy(acc.at[pl.ds(s * rpt, rpt)],
                        out_hbm.at[c, pl.ds(s * rpt, rpt)])

    return scatter_k


@functools.lru_cache(maxsize=None)
def _gather_k(R):
    return _make_gather(R, DT)


@functools.lru_cache(maxsize=None)
def _scatter_k(R):
    return _make_scatter(R, DS)


def _gather_piece(tab, idx3, R):
    return _gather_k(R)(tab, idx3)


def _scatter_piece(vals, idx3, zeros):
    return _scatter_k(vals.shape[0])(vals, idx3, zeros)


def _full(shape):
    return pl.BlockSpec(shape, lambda i: tuple(0 for _ in shape))


def _rows(b, d):
    return pl.BlockSpec((b, d), lambda i: (i, 0))


def _tc_call(body, grid, in_specs, out_specs, out_shapes):
    return pl.pallas_call(
        body,
        grid=(grid,),
        in_specs=in_specs,
        out_specs=out_specs,
        out_shape=out_shapes,
        compiler_params=pltpu.CompilerParams(
            dimension_semantics=("arbitrary",)),
    )


def _emb_body(nf, pos, vel, W, b, out):
    h = jnp.dot(nf[...], W[...], preferred_element_type=jnp.float32) + b[...]
    z = jnp.zeros((h.shape[0], DT - HID - 6), jnp.float32)
    out[...] = jnp.concatenate([h, pos[...], vel[...], z], axis=1)


def _init_edge_body(gr, gc, W1h, W1c, W1r, b1, W2, b2, scat, geom):
    be = gr.shape[0]
    hr = gr[:, 0:HID]
    hc = gc[:, 0:HID]
    mT = (gr[:, HID:HID + 8] - gc[:, HID:HID + 8]).T
    relT = mT[0:3]
    dvT = mT[3:6]
    r2T = relT[0:1] * relT[0:1] + relT[1:2] * relT[1:2] + relT[2:3] * relT[2:3]
    rT = jnp.sqrt(r2T)
    xT = rT * np.float32(1.0 / CUT)
    nT = np.float32(np.pi) * (
        lax.broadcasted_iota(jnp.int32, (RAD, 1), 0).astype(jnp.float32)
        + 1.0)
    sbT = (np.float32(np.sqrt(2.0 / CUT)) * jnp.sin(nT * xT)
           / (rT + 1e-9))
    p = PENV
    envT = (1.0 - ((p + 1) * (p + 2) / 2.0) * xT ** p
            + p * (p + 2) * xT ** (p + 1)
            - (p * (p + 1) / 2.0) * xT ** (p + 2))
    envT = jnp.where(xT < 1.0, envT, 0.0)
    radialT = sbT * envT
    uT = relT / (rT + 1e-9)
    ux, uy, uz = uT[0:1], uT[1:2], uT[2:3]
    c3 = np.float32(np.sqrt(3.0))
    c15 = np.float32(np.sqrt(15.0))
    c5 = np.float32(np.sqrt(5.0))
    YT = jnp.concatenate(
        [jnp.ones_like(ux), c3 * ux, c3 * uy, c3 * uz,
         c15 * ux * uy, c15 * uy * uz, (c5 / 2.0) * (3.0 * uz * uz - 1.0),
         c15 * ux * uz, (c15 / 2.0) * (ux * ux - uy * uy)], axis=0)
    hh = _silu(jnp.dot(hr, W1h[...], preferred_element_type=jnp.float32)
               + jnp.dot(hc, W1c[...], preferred_element_type=jnp.float32)
               + _dotT(radialT, W1r[...])
               + b1[...])
    g = jnp.dot(hh, W2[...], preferred_element_type=jnp.float32) + b2[...]
    egT = lax.dot_general(_expand_mat(), g, (((0,), (1,)), ((), ())),
                          preferred_element_type=jnp.float32)
    sT = jnp.concatenate([YT * egT, jnp.ones((1, be), jnp.float32)], axis=0)
    scat[:, 0:SHD + 1] = sT.T
    scat[:, SHD + 1:] = jnp.zeros((be, D0 - SHD - 1), jnp.float32)
    geom[...] = jnp.concatenate(
        [mT[0:6], radialT, jnp.zeros((DG - 22, be), jnp.float32)], axis=0)


def _init_node_body(t0, p0, p1, p2, p3, t1):
    ssum = p0[...] + p1[...] + p2[...] + p3[...]
    cnt = jnp.maximum(ssum[:, SHD:SHD + 1], 1.0)
    sh0 = ssum[:, 0:SHD] / cnt
    z = jnp.zeros((sh0.shape[0], DT - HID - SHD), jnp.float32)
    t1[...] = jnp.concatenate([t0[:, 0:HID], sh0, z], axis=1)


def _layer_edge_body(gr, gc, geom, W1h, W1c, W1r, W1i, b1, W2m, b2m,
                     Wg1, bg1, Wg2, bg2, out):
    be = gr.shape[0]
    hr = gr[:, 0:HID]
    hc = gc[:, 0:HID]
    shrT = _t_rows(gr[:, HID:HID + SHD], SHD)
    shcT = _t_rows(gc[:, HID:HID + SHD], SHD)
    relT = geom[0:3]
    dvT = geom[3:6]
    radialT = geom[6:6 + RAD]
    prodT = shrT * shcT
    ip0 = prodT[0:1]
    ip1 = prodT[1:2] + prodT[2:3] + prodT[3:4]
    ip2 = (prodT[4:5] + prodT[5:6] + prodT[6:7] + prodT[7:8] + prodT[8:9])
    shipT = jnp.concatenate([ip0, ip1, ip2], axis=0)
    h1 = _silu(jnp.dot(hr, W1h[...], preferred_element_type=jnp.float32)
               + jnp.dot(hc, W1c[...], preferred_element_type=jnp.float32)
               + _dotT(radialT, W1r[...])
               + _dotT(shipT, W1i[...])
               + b1[...])
    msg = _silu(jnp.dot(h1, W2m[...], preferred_element_type=jnp.float32)
                + b2m[...])
    hidg = _silu(jnp.dot(msg, Wg1[...], preferred_element_type=jnp.float32)
                 + bg1[...])
    g_all = jnp.dot(hidg, Wg2[...], preferred_element_type=jnp.float32) \
        + bg2[...]
    gT = _t_rows(g_all, 7)
    evpT = gT[0:1] * relT + gT[1:2] * dvT
    evvT = gT[2:3] * dvT + gT[3:4] * relT
    egT = lax.dot_general(_expand_mat(), gT[4:7], (((0,), (0,)), ((), ())),
                          preferred_element_type=jnp.float32)
    dshT = (shrT - shcT) * egT
    tailT = jnp.concatenate(
        [evpT, evvT, dshT, jnp.ones((1, be), jnp.float32)], axis=0)
    out[:, 0:HID] = msg
    out[:, HID:CNT + 1] = _t_cols(tailT, CNT + 1 - HID)
    out[:, CNT + 1:] = jnp.zeros((be, DS - CNT - 1), jnp.float32)


def _layer_node_body(t, p0, p1, p2, p3, pd, vd, Wa, Wb, b1, W2, b2,
                     tn, pdn, vdn):
    ssum = p0[...] + p1[...] + p2[...] + p3[...]
    inv = 1.0 / jnp.maximum(ssum[:, CNT:CNT + 1], 1.0)
    msg_agg = ssum[:, 0:HID] * inv
    pos_agg = ssum[:, HID:HID + 3] * inv
    vel_agg = ssum[:, HID + 3:HID + 6] * inv
    sh_agg = ssum[:, HID + 6:HID + 6 + SHD] * inv
    h = t[:, 0:HID]
    sh = t[:, HID:HID + SHD]
    hh = _silu(jnp.dot(h, Wa[...], preferred_element_type=jnp.float32)
               + jnp.dot(msg_agg, Wb[...], preferred_element_type=jnp.float32)
               + b1[...])
    hn = jnp.dot(hh, W2[...], preferred_element_type=jnp.float32) + b2[...]
    z = jnp.zeros((hn.shape[0], DT - HID - SHD), jnp.float32)
    tn[...] = jnp.concatenate([hn, sh + sh_agg, z], axis=1)
    pdn[...] = pd[...] + pos_agg
    vdn[...] = vd[...] + vel_agg


def _head_body(t, pos, vel, pd, vd, Wp1h, Wp1d, bp1, Wp2, bp2,
               Wv1h, Wv1d, Wv1v, bv1, Wv2, bv2, out):
    h = t[:, 0:HID]
    ph = _silu(jnp.dot(h, Wp1h[...], preferred_element_type=jnp.float32)
               + jnp.dot(pd[...], Wp1d[...],
                         preferred_element_type=jnp.float32) + bp1[...])
    pos_out = pos[...] + (jnp.dot(ph, Wp2[...],
                                  preferred_element_type=jnp.float32)
                          + bp2[...])
    vh = _silu(jnp.dot(h, Wv1h[...], preferred_element_type=jnp.float32)
               + jnp.dot(vd[...], Wv1d[...],
                         preferred_element_type=jnp.float32)
               + jnp.dot(vel[...], Wv1v[...],
                         preferred_element_type=jnp.float32) + bv1[...])
    vel_out = (jnp.dot(vh, Wv2[...], preferred_element_type=jnp.float32)
               + bv2[...])
    out[...] = jnp.concatenate([pos_out, vel_out], axis=1)


def _b2(b):
    return b.reshape(1, -1)


def kernel(node_feat, pos, vel, edge_index, params):
    f32 = jnp.float32
    npad = NP - N
    nf = jnp.pad(node_feat.astype(f32), ((0, npad), (0, 0)))
    posp = jnp.pad(pos.astype(f32), ((0, npad), (0, 0)))
    velp = jnp.pad(vel.astype(f32), ((0, npad), (0, 0)))

    ei = edge_index.astype(jnp.int32)
    row = ei[0]
    idx_g = [ei[:, o:o + s].reshape(NW, 2 * s // NW // CH, CH)
             for o, s in zip(OFFS, SPLITS)]
    idx_s = [row[o:o + s].reshape(NW, s // NW // CH, CH)
             for o, s in zip(OFFS, SPLITS)]

    zero_n0 = jnp.zeros((NP, D0), f32)
    zero_ns = jnp.zeros((NP, DS), f32)

    ngrid = NP // BN

    t0 = _tc_call(
        _emb_body, ngrid,
        [_rows(BN, DIN), _rows(BN, 3), _rows(BN, 3),
         _full((DIN, HID)), _full((1, HID))],
        _rows(BN, DT),
        jax.ShapeDtypeStruct((NP, DT), f32),
    )(nf, posp, velp, params['emb_W'], _b2(params['emb_b']))

    w = params['sh_init']

    def init_edge(gh, sz):
        return _tc_call(
            _init_edge_body, sz // BE,
            [_rows(BE, DT), _rows(BE, DT),
             _full((HID, HID)), _full((HID, HID)), _full((RAD, HID)),
             _full((1, HID)), _full((HID, 3)), _full((1, 3))],
            [_rows(BE, D0), pl.BlockSpec((DG, BE), lambda i: (0, i))],
            [jax.ShapeDtypeStruct((sz, D0), f32),
             jax.ShapeDtypeStruct((DG, sz), f32)],
        )(gh[:sz], gh[sz:], w['W1'][0:HID], w['W1'][HID:2 * HID],
          w['W1'][2 * HID:], _b2(w['b1']), w['W2'], _b2(w['b2']))

    g0 = _gather_piece(t0, idx_g[0], 2 * SPLITS[0])
    scat0a, geom0 = init_edge(g0, SPLITS[0])
    g1 = _gather_piece(t0, idx_g[1], 2 * SPLITS[1])
    scat0b, geom1 = init_edge(g1, SPLITS[1])
    pa = _scatter_piece(scat0a, idx_s[0], zero_n0)
    pb = _scatter_piece(scat0b, idx_s[1], zero_n0)
    t = _tc_call(
        _init_node_body, ngrid,
        [_rows(BN, DT), _rows(BN, D0), _rows(BN, D0),
         _rows(BN, D0), _rows(BN, D0)],
        _rows(BN, DT),
        jax.ShapeDtypeStruct((NP, DT), f32),
    )(t0, pa[0], pa[1], pb[0], pb[1])

    pd = jnp.zeros((NP, 3), f32)
    vd = jnp.zeros((NP, 3), f32)

    for lp in params['layers']:
        m = lp['msg']
        Wg1 = jnp.concatenate([lp['pos_basis']['W1'], lp['vel_basis']['W1'],
                               lp['mlp_sh']['W1']], axis=1)
        bg1 = jnp.concatenate([lp['pos_basis']['b1'], lp['vel_basis']['b1'],
                               lp['mlp_sh']['b1']])
        z64 = jnp.zeros((HID, 2), jnp.float32)
        z64b = jnp.zeros((HID, 3), jnp.float32)
        Wg2 = jnp.concatenate([
            jnp.concatenate([lp['pos_basis']['W2'], z64, z64b], axis=1),
            jnp.concatenate([z64, lp['vel_basis']['W2'], z64b], axis=1),
            jnp.concatenate([z64, z64, lp['mlp_sh']['W2']], axis=1),
        ], axis=0)
        bg2 = jnp.concatenate([lp['pos_basis']['b2'], lp['vel_basis']['b2'],
                               lp['mlp_sh']['b2']])
        def layer_edge(gh, geomh, sz):
            return _tc_call(
                _layer_edge_body, sz // BE,
                [_rows(BE, DT), _rows(BE, DT),
                 pl.BlockSpec((DG, BE), lambda i: (0, i)),
                 _full((HID, HID)), _full((HID, HID)), _full((RAD, HID)),
                 _full((3, HID)), _full((1, HID)),
                 _full((HID, HID)), _full((1, HID)),
                 _full((HID, 3 * HID)), _full((1, 3 * HID)),
                 _full((3 * HID, 7)), _full((1, 7))],
                _rows(BE, DS),
                jax.ShapeDtypeStruct((sz, DS), f32),
            )(gh[:sz], gh[sz:], geomh,
              m['W1'][0:HID], m['W1'][HID:2 * HID],
              m['W1'][2 * HID:2 * HID + RAD], m['W1'][2 * HID + RAD:],
              _b2(m['b1']), m['W2'], _b2(m['b2']),
              Wg1, _b2(bg1), Wg2, _b2(bg2))

        g0 = _gather_piece(t, idx_g[0], 2 * SPLITS[0])
        s0 = layer_edge(g0, geom0, SPLITS[0])
        g1 = _gather_piece(t, idx_g[1], 2 * SPLITS[1])
        s1 = layer_edge(g1, geom1, SPLITS[1])
        pa = _scatter_piece(s0, idx_s[0], zero_ns)
        pb = _scatter_piece(s1, idx_s[1], zero_ns)
        nw = lp['node_feat']
        t, pd, vd = _tc_call(
            _layer_node_body, ngrid,
            [_rows(BN, DT), _rows(BN, DS), _rows(BN, DS),
             _rows(BN, DS), _rows(BN, DS),
             _rows(BN, 3), _rows(BN, 3),
             _full((HID, HID)), _full((HID, HID)), _full((1, HID)),
             _full((HID, HID)), _full((1, HID))],
            [_rows(BN, DT), _rows(BN, 3), _rows(BN, 3)],
            [jax.ShapeDtypeStruct((NP, DT), f32),
             jax.ShapeDtypeStruct((NP, 3), f32),
             jax.ShapeDtypeStruct((NP, 3), f32)],
        )(t, pa[0], pa[1], pb[0], pb[1], pd, vd,
          nw['W1'][0:HID], nw['W1'][HID:], _b2(nw['b1']),
          nw['W2'], _b2(nw['b2']))

    ph = params['pos_head']
    vh = params['vel_head']
    out = _tc_call(
        _head_body, ngrid,
        [_rows(BN, DT), _rows(BN, 3), _rows(BN, 3),
         _rows(BN, 3), _rows(BN, 3),
         _full((HID, HID)), _full((3, HID)), _full((1, HID)),
         _full((HID, 3)), _full((1, 3)),
         _full((HID, HID)), _full((3, HID)), _full((3, HID)),
         _full((1, HID)), _full((HID, 3)), _full((1, 3))],
        _rows(BN, 6),
        jax.ShapeDtypeStruct((NP, 6), f32),
    )(t, posp, velp, pd, vd,
      ph['W1'][0:HID], ph['W1'][HID:], _b2(ph['b1']), ph['W2'], _b2(ph['b2']),
      vh['W1'][0:HID], vh['W1'][HID:HID + 3], vh['W1'][HID + 3:],
      _b2(vh['b1']), vh['W2'], _b2(vh['b2']))

    return out[:N]


# speedup vs baseline: 5.5276x; 1.0412x over previous
import functools

import jax
import jax.numpy as jnp
import numpy as np
from jax import lax
from jax.experimental import pallas as pl
from jax.experimental.pallas import tpu as pltpu
from jax.experimental.pallas import tpu_sc as plsc

N = 10000
NP = 10240
E = 320000
DIN = 128
HID = 64
RAD = 16
SHD = 9
CUT = 5.0
PENV = 5

NC = 2
NS = 16
NW = NC * NS
CH = 80

DT = 128
DS = 128
D0 = 128
CNT = 79
DG = 24

BE = 6400
BN = 2048

SPLITS = (102400, 115200, 102400)
OFFS = (0, 102400, 217600)


def _silu(x):
    return x * jax.nn.sigmoid(x)


def _expand_mat():
    ii = lax.broadcasted_iota(jnp.int32, (3, SHD), 0)
    jj = lax.broadcasted_iota(jnp.int32, (3, SHD), 1)
    return ((jj >= ii * ii) & (jj < (ii + 1) * (ii + 1))).astype(jnp.float32)


def _expand_deg(g):
    return jnp.dot(g, _expand_mat(), preferred_element_type=jnp.float32)


def _dotT(at, w):
    return lax.dot_general(at, w, (((0,), (0,)), ((), ())),
                           preferred_element_type=jnp.float32)


def _eye(k):
    return (lax.broadcasted_iota(jnp.int32, (k, k), 0) ==
            lax.broadcasted_iota(jnp.int32, (k, k), 1)).astype(jnp.float32)


def _t_rows(a, k):
    return lax.dot_general(_eye(k), a, (((1,), (1,)), ((), ())),
                           preferred_element_type=jnp.float32)


def _t_cols(at, k):
    return lax.dot_general(at, _eye(k), (((0,), (0,)), ((), ())),
                           preferred_element_type=jnp.float32)


def _make_gather(R, D, ch=CH):
    per_w = R // NW
    nch = per_w // ch
    nsup = nch // 2
    SUP = 2 * ch
    mesh = plsc.VectorSubcoreMesh(core_axis_name="c", subcore_axis_name="s")

    @functools.partial(
        pl.kernel, mesh=mesh,
        out_type=jax.ShapeDtypeStruct((R, D), jnp.float32),
        scratch_types=[
            pltpu.VMEM((nch, ch), jnp.int32),
            pltpu.VMEM((SUP, D), jnp.float32),
            pltpu.VMEM((SUP, D), jnp.float32),
            pltpu.SemaphoreType.DMA,
            pltpu.SemaphoreType.DMA,
            pltpu.SemaphoreType.DMA,
            pltpu.SemaphoreType.DMA,
        ],
    )
    def gather_k(tab_hbm, idx_hbm, out_hbm, idx_v,
                 buf0, buf1, in0, in1, ou0, ou1):
        wid = lax.axis_index("s") * NC + lax.axis_index("c")
        pltpu.sync_copy(idx_hbm.at[wid], idx_v)
        bufs = (buf0, buf1)
        ins = (in0, in1)
        ous = (ou0, ou1)

        def start_super(s, b):
            for h in range(2):
                pltpu.async_copy(tab_hbm.at[idx_v.at[2 * s + h]],
                                 bufs[b].at[pl.ds(h * ch, ch)], ins[b])

        def wait_in(b):
            for h in range(2):
                pltpu.make_async_copy(tab_hbm.at[pl.ds(0, ch)],
                                      bufs[b].at[pl.ds(0, ch)],
                                      ins[b]).wait()

        def wait_out(b):
            pltpu.make_async_copy(bufs[b], out_hbm.at[pl.ds(0, SUP)],
                                  ous[b]).wait()

        start_super(0, 0)

        def outer(jj, carry):
            for k in range(2):
                s = jj * 2 + k
                nb = 1 - k

                @pl.when(s + 1 < nsup)
                def _ahead():
                    @pl.when(s >= 1)
                    def _drain():
                        wait_out(nb)

                    start_super(s + 1, nb)

                wait_in(k)
                pltpu.async_copy(
                    bufs[k], out_hbm.at[pl.ds(wid * per_w + s * SUP, SUP)],
                    ous[k])
            return carry

        lax.fori_loop(0, nsup // 2, outer, 0)
        if nsup % 2 == 1:
            wait_in(0)
            pltpu.async_copy(
                bufs[0],
                out_hbm.at[pl.ds(wid * per_w + (nsup - 1) * SUP, SUP)],
                ous[0])
        wait_out(1 - (nsup % 2))
        wait_out(nsup % 2)

    return gather_k


def _make_scatter(R, D, ch=CH):
    per_w = R // NW
    nch = per_w // ch
    rpt = NP // NS
    mesh = plsc.VectorSubcoreMesh(core_axis_name="c", subcore_axis_name="s")

    @functools.partial(
        pl.kernel, mesh=mesh,
        out_type=jax.ShapeDtypeStruct((NC, NP, D), jnp.float32),
        scratch_types=[
            pltpu.VMEM((nch, ch), jnp.int32),
            pltpu.VMEM((ch, D), jnp.float32),
            pltpu.VMEM((ch, D), jnp.float32),
            pltpu.VMEM_SHARED((NP, D), jnp.float32),
            pltpu.SemaphoreType.DMA,
            pltpu.SemaphoreType.DMA,
            pltpu.SemaphoreType.DMA,
            pltpu.SemaphoreType.DMA,
        ],
    )
    def scatter_k(val_hbm, idx_hbm, zero_hbm, out_hbm, idx_v,
                  buf0, buf1, acc, ld0, ld1, ad0, ad1):
        c = lax.axis_index("c")
        s = lax.axis_index("s")
        wid = s * NC + c
        pltpu.sync_copy(zero_hbm.at[pl.ds(s * rpt, rpt)],
                        acc.at[pl.ds(s * rpt, rpt)])
        pltpu.sync_copy(idx_hbm.at[wid], idx_v)
        plsc.subcore_barrier()
        bufs = (buf0, buf1)
        lds = (ld0, ld1)
        ads = (ad0, ad1)

        def load(j, b):
            pltpu.async_copy(val_hbm.at[pl.ds(wid * per_w + j * ch, ch)],
                             bufs[b], lds[b])

        def wait_ld(b):
            pltpu.make_async_copy(val_hbm.at[pl.ds(0, ch)], bufs[b],
                                  lds[b]).wait()

        def wait_add(b):
            pltpu.make_async_copy(val_hbm.at[pl.ds(0, ch)], bufs[b],
                                  ads[b]).wait()

        load(0, 0)

        def outer(jj, carry):
            for k in range(2):
                j = jj * 2 + k
                nb = 1 - k

                @pl.when(j + 1 < nch)
                def _ahead():
                    @pl.when(j >= 1)
                    def _drain():
                        wait_add(nb)

                    load(j + 1, nb)

                wait_ld(k)
                pltpu.async_copy(bufs[k], acc.at[idx_v.at[j]], ads[k],
                                 add=True)
            return carry

        lax.fori_loop(0, nch // 2, outer, 0)
        if nch % 2 == 1:
            wait_ld(0)
            pltpu.async_copy(bufs[0], acc.at[idx_v.at[nch - 1]], ads[0],
                             add=True)
        wait_add(1 - (nch % 2))
        wait_add(nch % 2)
        plsc.subcore_barrier()
        pltpu.sync_copy(acc.at[pl.ds(s * rpt, rpt)],
                        out_hbm.at[c, pl.ds(s * rpt, rpt)])

    return scatter_k


@functools.lru_cache(maxsize=None)
def _gather_k(R):
    return _make_gather(R, DT)


@functools.lru_cache(maxsize=None)
def _scatter_k(R):
    return _make_scatter(R, DS)


def _gather_piece(tab, idx3, R):
    return _gather_k(R)(tab, idx3)


def _scatter_piece(vals, idx3, zeros):
    return _scatter_k(vals.shape[0])(vals, idx3, zeros)


def _full(shape):
    return pl.BlockSpec(shape, lambda i: tuple(0 for _ in shape))


def _rows(b, d):
    return pl.BlockSpec((b, d), lambda i: (i, 0))


def _tc_call(body, grid, in_specs, out_specs, out_shapes):
    return pl.pallas_call(
        body,
        grid=(grid,),
        in_specs=in_specs,
        out_specs=out_specs,
        out_shape=out_shapes,
        compiler_params=pltpu.CompilerParams(
            dimension_semantics=("arbitrary",)),
    )


def _emb_body(nf, pos, vel, W, b, out):
    h = jnp.dot(nf[...], W[...], preferred_element_type=jnp.float32) + b[...]
    z = jnp.zeros((h.shape[0], DT - HID - 6), jnp.float32)
    out[...] = jnp.concatenate([h, pos[...], vel[...], z], axis=1)


def _init_edge_body(gr, gc, W1h, W1c, W1r, b1, W2, b2, scat, geom):
    be = gr.shape[0]
    hr = gr[:, 0:HID]
    hc = gc[:, 0:HID]
    mT = (gr[:, HID:HID + 8] - gc[:, HID:HID + 8]).T
    relT = mT[0:3]
    dvT = mT[3:6]
    r2T = relT[0:1] * relT[0:1] + relT[1:2] * relT[1:2] + relT[2:3] * relT[2:3]
    rT = jnp.sqrt(r2T)
    xT = rT * np.float32(1.0 / CUT)
    nT = np.float32(np.pi) * (
        lax.broadcasted_iota(jnp.int32, (RAD, 1), 0).astype(jnp.float32)
        + 1.0)
    sbT = (np.float32(np.sqrt(2.0 / CUT)) * jnp.sin(nT * xT)
           / (rT + 1e-9))
    p = PENV
    envT = (1.0 - ((p + 1) * (p + 2) / 2.0) * xT ** p
            + p * (p + 2) * xT ** (p + 1)
            - (p * (p + 1) / 2.0) * xT ** (p + 2))
    envT = jnp.where(xT < 1.0, envT, 0.0)
    radialT = sbT * envT
    uT = relT / (rT + 1e-9)
    ux, uy, uz = uT[0:1], uT[1:2], uT[2:3]
    c3 = np.float32(np.sqrt(3.0))
    c15 = np.float32(np.sqrt(15.0))
    c5 = np.float32(np.sqrt(5.0))
    YT = jnp.concatenate(
        [jnp.ones_like(ux), c3 * ux, c3 * uy, c3 * uz,
         c15 * ux * uy, c15 * uy * uz, (c5 / 2.0) * (3.0 * uz * uz - 1.0),
         c15 * ux * uz, (c15 / 2.0) * (ux * ux - uy * uy)], axis=0)
    hh = _silu(jnp.dot(hr, W1h[...], preferred_element_type=jnp.float32)
               + jnp.dot(hc, W1c[...], preferred_element_type=jnp.float32)
               + _dotT(radialT, W1r[...])
               + b1[...])
    g = jnp.dot(hh, W2[...], preferred_element_type=jnp.float32) + b2[...]
    egT = lax.dot_general(_expand_mat(), g, (((0,), (1,)), ((), ())),
                          preferred_element_type=jnp.float32)
    sT = jnp.concatenate([YT * egT, jnp.ones((1, be), jnp.float32)], axis=0)
    scat[:, 0:SHD + 1] = sT.T
    scat[:, SHD + 1:] = jnp.zeros((be, D0 - SHD - 1), jnp.float32)
    geom[...] = jnp.concatenate(
        [mT[0:6], radialT, jnp.zeros((DG - 22, be), jnp.float32)], axis=0)


def _init_node_body(t0, p0, p1, p2, p3, p4, p5, t1):
    ssum = (p0[...] + p1[...] + p2[...] + p3[...] + p4[...] + p5[...])
    cnt = jnp.maximum(ssum[:, SHD:SHD + 1], 1.0)
    sh0 = ssum[:, 0:SHD] / cnt
    z = jnp.zeros((sh0.shape[0], DT - HID - SHD), jnp.float32)
    t1[...] = jnp.concatenate([t0[:, 0:HID], sh0, z], axis=1)


def _layer_edge_body(gr, gc, geom, W1h, W1c, W1r, W1i, b1, W2m, b2m,
                     Wg1, bg1, Wg2, bg2, out):
    be = gr.shape[0]
    hr = gr[:, 0:HID]
    hc = gc[:, 0:HID]
    shrT = _t_rows(gr[:, HID:HID + SHD], SHD)
    shcT = _t_rows(gc[:, HID:HID + SHD], SHD)
    relT = geom[0:3]
    dvT = geom[3:6]
    radialT = geom[6:6 + RAD]
    prodT = shrT * shcT
    ip0 = prodT[0:1]
    ip1 = prodT[1:2] + prodT[2:3] + prodT[3:4]
    ip2 = (prodT[4:5] + prodT[5:6] + prodT[6:7] + prodT[7:8] + prodT[8:9])
    shipT = jnp.concatenate([ip0, ip1, ip2], axis=0)
    h1 = _silu(jnp.dot(hr, W1h[...], preferred_element_type=jnp.float32)
               + jnp.dot(hc, W1c[...], preferred_element_type=jnp.float32)
               + _dotT(radialT, W1r[...])
               + _dotT(shipT, W1i[...])
               + b1[...])
    msg = _silu(jnp.dot(h1, W2m[...], preferred_element_type=jnp.float32)
                + b2m[...])
    hidg = _silu(jnp.dot(msg, Wg1[...], preferred_element_type=jnp.float32)
                 + bg1[...])
    g_all = jnp.dot(hidg, Wg2[...], preferred_element_type=jnp.float32) \
        + bg2[...]
    gT = _t_rows(g_all, 7)
    evpT = gT[0:1] * relT + gT[1:2] * dvT
    evvT = gT[2:3] * dvT + gT[3:4] * relT
    egT = lax.dot_general(_expand_mat(), gT[4:7], (((0,), (0,)), ((), ())),
                          preferred_element_type=jnp.float32)
    dshT = (shrT - shcT) * egT
    tailT = jnp.concatenate(
        [evpT, evvT, dshT, jnp.ones((1, be), jnp.float32)], axis=0)
    out[:, 0:HID] = msg
    out[:, HID:CNT + 1] = _t_cols(tailT, CNT + 1 - HID)
    out[:, CNT + 1:] = jnp.zeros((be, DS - CNT - 1), jnp.float32)


def _layer_node_body(t, p0, p1, p2, p3, p4, p5, pd, vd, Wa, Wb, b1, W2, b2,
                     tn, pdn, vdn):
    ssum = (p0[...] + p1[...] + p2[...] + p3[...] + p4[...] + p5[...])
    inv = 1.0 / jnp.maximum(ssum[:, CNT:CNT + 1], 1.0)
    msg_agg = ssum[:, 0:HID] * inv
    pos_agg = ssum[:, HID:HID + 3] * inv
    vel_agg = ssum[:, HID + 3:HID + 6] * inv
    sh_agg = ssum[:, HID + 6:HID + 6 + SHD] * inv
    h = t[:, 0:HID]
    sh = t[:, HID:HID + SHD]
    hh = _silu(jnp.dot(h, Wa[...], preferred_element_type=jnp.float32)
               + jnp.dot(msg_agg, Wb[...], preferred_element_type=jnp.float32)
               + b1[...])
    hn = jnp.dot(hh, W2[...], preferred_element_type=jnp.float32) + b2[...]
    z = jnp.zeros((hn.shape[0], DT - HID - SHD), jnp.float32)
    tn[...] = jnp.concatenate([hn, sh + sh_agg, z], axis=1)
    pdn[...] = pd[...] + pos_agg
    vdn[...] = vd[...] + vel_agg


def _head_body(t, pos, vel, pd, vd, Wp1h, Wp1d, bp1, Wp2, bp2,
               Wv1h, Wv1d, Wv1v, bv1, Wv2, bv2, out):
    h = t[:, 0:HID]
    ph = _silu(jnp.dot(h, Wp1h[...], preferred_element_type=jnp.float32)
               + jnp.dot(pd[...], Wp1d[...],
                         preferred_element_type=jnp.float32) + bp1[...])
    pos_out = pos[...] + (jnp.dot(ph, Wp2[...],
                                  preferred_element_type=jnp.float32)
                          + bp2[...])
    vh = _silu(jnp.dot(h, Wv1h[...], preferred_element_type=jnp.float32)
               + jnp.dot(vd[...], Wv1d[...],
                         preferred_element_type=jnp.float32)
               + jnp.dot(vel[...], Wv1v[...],
                         preferred_element_type=jnp.float32) + bv1[...])
    vel_out = (jnp.dot(vh, Wv2[...], preferred_element_type=jnp.float32)
               + bv2[...])
    out[...] = jnp.concatenate([pos_out, vel_out], axis=1)


def _b2(b):
    return b.reshape(1, -1)


def kernel(node_feat, pos, vel, edge_index, params):
    f32 = jnp.float32
    npad = NP - N
    nf = jnp.pad(node_feat.astype(f32), ((0, npad), (0, 0)))
    posp = jnp.pad(pos.astype(f32), ((0, npad), (0, 0)))
    velp = jnp.pad(vel.astype(f32), ((0, npad), (0, 0)))

    ei = edge_index.astype(jnp.int32)
    row = ei[0]
    idx_g = [ei[:, o:o + s].reshape(NW, 2 * s // NW // CH, CH)
             for o, s in zip(OFFS, SPLITS)]
    idx_s = [row[o:o + s].reshape(NW, s // NW // CH, CH)
             for o, s in zip(OFFS, SPLITS)]

    zero_n0 = jnp.zeros((NP, D0), f32)
    zero_ns = jnp.zeros((NP, DS), f32)

    ngrid = NP // BN

    t0 = _tc_call(
        _emb_body, ngrid,
        [_rows(BN, DIN), _rows(BN, 3), _rows(BN, 3),
         _full((DIN, HID)), _full((1, HID))],
        _rows(BN, DT),
        jax.ShapeDtypeStruct((NP, DT), f32),
    )(nf, posp, velp, params['emb_W'], _b2(params['emb_b']))

    w = params['sh_init']

    def init_edge(gh, sz):
        return _tc_call(
            _init_edge_body, sz // BE,
            [_rows(BE, DT), _rows(BE, DT),
             _full((HID, HID)), _full((HID, HID)), _full((RAD, HID)),
             _full((1, HID)), _full((HID, 3)), _full((1, 3))],
            [_rows(BE, D0), pl.BlockSpec((DG, BE), lambda i: (0, i))],
            [jax.ShapeDtypeStruct((sz, D0), f32),
             jax.ShapeDtypeStruct((DG, sz), f32)],
        )(gh[:sz], gh[sz:], w['W1'][0:HID], w['W1'][HID:2 * HID],
          w['W1'][2 * HID:], _b2(w['b1']), w['W2'], _b2(w['b2']))

    scats, geoms = [], []
    for h, sz in enumerate(SPLITS):
        gh = _gather_piece(t0, idx_g[h], 2 * sz)
        sc_h, ge_h = init_edge(gh, sz)
        scats.append(sc_h)
        geoms.append(ge_h)
    ps = [_scatter_piece(scats[h], idx_s[h], zero_n0)
          for h in range(len(SPLITS))]
    t = _tc_call(
        _init_node_body, ngrid,
        [_rows(BN, DT)] + [_rows(BN, D0)] * 6,
        _rows(BN, DT),
        jax.ShapeDtypeStruct((NP, DT), f32),
    )(t0, ps[0][0], ps[0][1], ps[1][0], ps[1][1], ps[2][0], ps[2][1])

    pd = jnp.zeros((NP, 3), f32)
    vd = jnp.zeros((NP, 3), f32)

    for lp in params['layers']:
        m = lp['msg']
        Wg1 = jnp.concatenate([lp['pos_basis']['W1'], lp['vel_basis']['W1'],
                               lp['mlp_sh']['W1']], axis=1)
        bg1 = jnp.concatenate([lp['pos_basis']['b1'], lp['vel_basis']['b1'],
                               lp['mlp_sh']['b1']])
        z64 = jnp.zeros((HID, 2), jnp.float32)
        z64b = jnp.zeros((HID, 3), jnp.float32)
        Wg2 = jnp.concatenate([
            jnp.concatenate([lp['pos_basis']['W2'], z64, z64b], axis=1),
            jnp.concatenate([z64, lp['vel_basis']['W2'], z64b], axis=1),
            jnp.concatenate([z64, z64, lp['mlp_sh']['W2']], axis=1),
        ], axis=0)
        bg2 = jnp.concatenate([lp['pos_basis']['b2'], lp['vel_basis']['b2'],
                               lp['mlp_sh']['b2']])
        def layer_edge(gh, geomh, sz):
            return _tc_call(
                _layer_edge_body, sz // BE,
                [_rows(BE, DT), _rows(BE, DT),
                 pl.BlockSpec((DG, BE), lambda i: (0, i)),
                 _full((HID, HID)), _full((HID, HID)), _full((RAD, HID)),
                 _full((3, HID)), _full((1, HID)),
                 _full((HID, HID)), _full((1, HID)),
                 _full((HID, 3 * HID)), _full((1, 3 * HID)),
                 _full((3 * HID, 7)), _full((1, 7))],
                _rows(BE, DS),
                jax.ShapeDtypeStruct((sz, DS), f32),
            )(gh[:sz], gh[sz:], geomh,
              m['W1'][0:HID], m['W1'][HID:2 * HID],
              m['W1'][2 * HID:2 * HID + RAD], m['W1'][2 * HID + RAD:],
              _b2(m['b1']), m['W2'], _b2(m['b2']),
              Wg1, _b2(bg1), Wg2, _b2(bg2))

        es = []
        for h, sz in enumerate(SPLITS):
            gh = _gather_piece(t, idx_g[h], 2 * sz)
            es.append(layer_edge(gh, geoms[h], sz))
        ps = [_scatter_piece(es[h], idx_s[h], zero_ns)
              for h in range(len(SPLITS))]
        nw = lp['node_feat']
        t, pd, vd = _tc_call(
            _layer_node_body, ngrid,
            [_rows(BN, DT)] + [_rows(BN, DS)] * 6 +
            [_rows(BN, 3), _rows(BN, 3),
             _full((HID, HID)), _full((HID, HID)), _full((1, HID)),
             _full((HID, HID)), _full((1, HID))],
            [_rows(BN, DT), _rows(BN, 3), _rows(BN, 3)],
            [jax.ShapeDtypeStruct((NP, DT), f32),
             jax.ShapeDtypeStruct((NP, 3), f32),
             jax.ShapeDtypeStruct((NP, 3), f32)],
        )(t, ps[0][0], ps[0][1], ps[1][0], ps[1][1], ps[2][0], ps[2][1],
          pd, vd,
          nw['W1'][0:HID], nw['W1'][HID:], _b2(nw['b1']),
          nw['W2'], _b2(nw['b2']))

    ph = params['pos_head']
    vh = params['vel_head']
    out = _tc_call(
        _head_body, ngrid,
        [_rows(BN, DT), _rows(BN, 3), _rows(BN, 3),
         _rows(BN, 3), _rows(BN, 3),
         _full((HID, HID)), _full((3, HID)), _full((1, HID)),
         _full((HID, 3)), _full((1, 3)),
         _full((HID, HID)), _full((3, HID)), _full((3, HID)),
         _full((1, HID)), _full((HID, 3)), _full((1, 3))],
        _rows(BN, 6),
        jax.ShapeDtypeStruct((NP, 6), f32),
    )(t, posp, velp, pd, vd,
      ph['W1'][0:HID], ph['W1'][HID:], _b2(ph['b1']), ph['W2'], _b2(ph['b2']),
      vh['W1'][0:HID], vh['W1'][HID:HID + 3], vh['W1'][HID + 3:],
      _b2(vh['b1']), vh['W2'], _b2(vh['b2']))

    return out[:N]
